# Initial kernel scaffold; baseline (speedup 1.0000x reference)
import jax
import jax.numpy as jnp
from jax.experimental import pallas as pl


def kernel(boxes_norm, row_embed, col_embed):
    raise NotImplementedError("write your pallas kernel here")


# speedup vs baseline: 1.0668x; 1.0668x over previous
import functools

import jax
import jax.numpy as jnp
from jax import lax
from jax.experimental import pallas as pl
from jax.experimental.pallas import tpu as pltpu
from jax.experimental.pallas import tpu_sc as plsc

_CHANNELS = 256
_GRID = 32
_N = 20000
_NPAD = 20480
_NW = 32
_BPW = _NPAD // _NW
_CHUNK = 128
_NCHUNK = _BPW // _CHUNK
_MAGIC = 12582912.0


@functools.partial(
    pl.kernel,
    mesh=plsc.VectorSubcoreMesh(core_axis_name="c", subcore_axis_name="s"),
    out_type=jax.ShapeDtypeStruct((_NPAD, _CHANNELS), jnp.float32),
    scratch_types=[
        pltpu.VMEM((4 * _BPW,), jnp.float32),
        pltpu.VMEM((_NCHUNK, _CHUNK), jnp.int32),
        pltpu.VMEM((_NCHUNK, _CHUNK), jnp.int32),
        pltpu.VMEM((_CHUNK, 128), jnp.float32),
        pltpu.VMEM((_CHUNK, 128), jnp.float32),
        pltpu.SemaphoreType.DMA,
        pltpu.SemaphoreType.DMA,
    ],
)
def _pos_enc_sc(boxes_hbm, table_hbm, out_hbm, boxes_v, cxi_v, cyi_v,
                rows0_v, rows1_v, sem0, sem1):
    wid = lax.axis_index("s") * 2 + lax.axis_index("c")
    box_base = wid * _BPW

    for i in range(4):
        pltpu.sync_copy(boxes_hbm.at[pl.ds(i * _NPAD + box_base, _BPW)],
                        boxes_v.at[pl.ds(i * _BPW, _BPW)])

    half = 0.5 * (_GRID - 1)
    for s in range(_BPW // 16):
        o = s * 16
        x1 = boxes_v[pl.ds(o, 16)]
        y1 = boxes_v[pl.ds(_BPW + o, 16)]
        x2 = boxes_v[pl.ds(2 * _BPW + o, 16)]
        y2 = boxes_v[pl.ds(3 * _BPW + o, 16)]
        cxr = ((x1 + x2) * half + _MAGIC) - _MAGIC
        cyr = ((y1 + y2) * half + _MAGIC) - _MAGIC
        cxi = jnp.clip(cxr, 0.0, _GRID - 1.0).astype(jnp.int32)
        cyi = jnp.clip(cyr, 0.0, _GRID - 1.0).astype(jnp.int32) + _GRID
        cxi_v[s // 8, pl.ds((s % 8) * 16, 16)] = cxi
        cyi_v[s // 8, pl.ds((s % 8) * 16, 16)] = cyi

    bufs = (rows0_v, rows1_v)
    sems = (sem0, sem1)
    jobs = []
    for c in range(_NCHUNK):
        jobs.append((cxi_v.at[c], 0, c))
        jobs.append((cyi_v.at[c], 128, c))
    copies = [None, None]
    dsts = [None, None]
    for j, (idx_ref, ch, c) in enumerate(jobs):
        b = j % 2
        if copies[b] is not None:
            copies[b].wait()
            pltpu.sync_copy(bufs[b], dsts[b])
        copies[b] = pltpu.async_copy(table_hbm.at[idx_ref], bufs[b], sems[b])
        dsts[b] = out_hbm.at[pl.ds(box_base + c * _CHUNK, _CHUNK),
                             pl.ds(ch, 128)]
    for b in (0, 1):
        copies[b].wait()
        pltpu.sync_copy(bufs[b], dsts[b])


def kernel(boxes_norm, row_embed, col_embed):
    boxes_t = jnp.pad(boxes_norm, ((0, _NPAD - _N), (0, 0))).T.reshape(-1)
    table = jnp.concatenate([col_embed, row_embed], axis=0)
    out = _pos_enc_sc(boxes_t, table)
    return out[:_N, :, None, None]
